# Initial kernel scaffold; baseline (speedup 1.0000x reference)
import jax
import jax.numpy as jnp
from jax.experimental import pallas as pl


def kernel(inputs, labels, index, average_center, features):
    raise NotImplementedError("write your pallas kernel here")


# speedup vs baseline: 20.2870x; 20.2870x over previous
import jax
import jax.numpy as jnp
from jax.experimental import pallas as pl
from jax.experimental.pallas import tpu as pltpu

_TEMP = 0.05
_TILE = 2000


def _hm_kernel(x_ref, lab_ref, f_ref, loss_ref, acc_ref, pick_ref):
    t = pl.program_id(0)
    nt = pl.num_programs(0)

    x = x_ref[...]
    nrm = jnp.sqrt(jnp.sum(x * x, axis=1, keepdims=True))
    x = x / (jnp.maximum(nrm, 1e-12) * _TEMP)

    f = f_ref[...]
    logits = jax.lax.dot_general(
        x, f, (((1,), (1,)), ((), ())), preferred_element_type=jnp.float32)
    e = jnp.exp(logits)
    s = jnp.sum(e, axis=1, keepdims=True)

    cols = t * _TILE + jax.lax.broadcasted_iota(jnp.int32, logits.shape, 1)
    hit = cols == lab_ref[...]
    p = jnp.sum(jnp.where(hit, logits, 0.0), axis=1, keepdims=True)

    @pl.when(t == 0)
    def _():
        acc_ref[...] = s
        pick_ref[...] = p

    @pl.when(t != 0)
    def _():
        acc_ref[...] += s
        pick_ref[...] += p

    @pl.when(t == nt - 1)
    def _():
        prob = jnp.exp(pick_ref[...]) / acc_ref[...]
        loss_ref[...] = -jnp.mean(jnp.log(prob + 1e-6),
                                  axis=(0, 1), keepdims=True)


def kernel(inputs, labels, index, average_center, features):
    B, nfeat = inputs.shape
    n = features.shape[0]
    labs = labels.astype(jnp.int32).reshape(B, 1)
    loss = pl.pallas_call(
        _hm_kernel,
        grid=(n // _TILE,),
        in_specs=[
            pl.BlockSpec((B, nfeat), lambda t: (0, 0)),
            pl.BlockSpec((B, 1), lambda t: (0, 0)),
            pl.BlockSpec((_TILE, nfeat), lambda t: (t, 0)),
        ],
        out_specs=pl.BlockSpec((1, 1), lambda t: (0, 0)),
        out_shape=jax.ShapeDtypeStruct((1, 1), jnp.float32),
        scratch_shapes=[
            pltpu.VMEM((B, 1), jnp.float32),
            pltpu.VMEM((B, 1), jnp.float32),
        ],
    )(inputs, labs, features)
    return loss[0, 0]
